# Initial kernel scaffold; baseline (speedup 1.0000x reference)
import jax
import jax.numpy as jnp
from jax.experimental import pallas as pl


def kernel(input_tokens, index, src_enc, cu_seqlens, cat_table, lemma_table, head_W, head_b, dep_W, dep_b):
    raise NotImplementedError("write your pallas kernel here")


# speedup vs baseline: 1.6144x; 1.6144x over previous
import functools

import jax
import jax.numpy as jnp
from jax import lax
from jax.experimental import pallas as pl
from jax.experimental.pallas import tpu as pltpu
from jax.experimental.pallas import tpu_sc as plsc

B = 4
SEG = 256
TOTAL = B * SEG
SRC_LEN = 128
CAT_D = 32
LEM_D = 128
RNN_D = 128
REL_D = 256
AMR_D = CAT_D + LEM_D
IN_D = AMR_D + RNN_D

NC = 2
NS = 16
NW = NC * NS

_MESH = plsc.VectorSubcoreMesh(core_axis_name="c", subcore_axis_name="s")


def _wid():
    return lax.axis_index("s") * NC + lax.axis_index("c")


TOK_PER_W = TOTAL // NW


@functools.partial(
    pl.kernel,
    out_type=(
        jax.ShapeDtypeStruct((TOTAL, 128), jnp.float32),
        jax.ShapeDtypeStruct((TOTAL, LEM_D), jnp.float32),
        jax.ShapeDtypeStruct((TOTAL, RNN_D), jnp.float32),
    ),
    mesh=_MESH,
    scratch_types=[
        pltpu.VMEM((TOK_PER_W,), jnp.int32),
        pltpu.VMEM((TOK_PER_W,), jnp.int32),
        pltpu.VMEM((TOK_PER_W,), jnp.int32),
        pltpu.VMEM((TOK_PER_W, 128), jnp.float32),
        pltpu.VMEM((TOK_PER_W, LEM_D), jnp.float32),
        pltpu.VMEM((TOK_PER_W, RNN_D), jnp.float32),
        pltpu.SemaphoreType.DMA,
    ],
)
def _sc_gather(cat_ids, lem_ids, head_idx, cat_table, lemma_table, src_flat,
               cat_out, lem_out, head_out,
               cidx_v, lidx_v, hidx_v, crows_v, lrows_v, hrows_v, sem):
    base = _wid() * TOK_PER_W
    pltpu.sync_copy(cat_ids.at[pl.ds(base, TOK_PER_W)], cidx_v)
    pltpu.sync_copy(lem_ids.at[pl.ds(base, TOK_PER_W)], lidx_v)
    pltpu.sync_copy(head_idx.at[pl.ds(base, TOK_PER_W)], hidx_v)
    pltpu.async_copy(cat_table.at[cidx_v], crows_v, sem).wait()
    pltpu.sync_copy(crows_v, cat_out.at[pl.ds(base, TOK_PER_W)])
    pltpu.async_copy(lemma_table.at[lidx_v], lrows_v, sem).wait()
    pltpu.sync_copy(lrows_v, lem_out.at[pl.ds(base, TOK_PER_W)])
    pltpu.async_copy(src_flat.at[hidx_v], hrows_v, sem).wait()
    pltpu.sync_copy(hrows_v, head_out.at[pl.ds(base, TOK_PER_W)])


I_PER_W = SEG // 8
JH = SEG // 2


@functools.partial(
    pl.kernel,
    out_type=jax.ShapeDtypeStruct((B * SEG * SEG, REL_D), jnp.float32),
    mesh=_MESH,
    scratch_types=[
        pltpu.VMEM((SEG,), jnp.int32),
        pltpu.VMEM((JH,), jnp.int32),
        pltpu.VMEM((JH, REL_D), jnp.float32),
        pltpu.VMEM((JH, REL_D), jnp.float32),
        pltpu.SemaphoreType.DMA,
    ],
)
def _sc_assemble(p_hbm, a_hbm, idx_hbm, out_hbm,
                 idx_v, gidx_v, a_v, g_v, sem):
    w = _wid()
    b = w // 8
    iblk = w % 8
    pltpu.sync_copy(idx_hbm.at[pl.ds(b * SEG, SEG)], idx_v)

    def do_half(h, _):
        pltpu.sync_copy(a_hbm.at[pl.ds(b * SEG + h * JH, JH)], a_v)

        def per_i(i, _):
            i_g = iblk * I_PER_W + i
            off = (b * SEG + i_g) * SRC_LEN

            def mk_idx(c, _):
                gidx_v[pl.ds(c * 16, 16)] = idx_v[pl.ds(h * JH + c * 16, 16)] + off
                return 0

            lax.fori_loop(0, JH // 16, mk_idx, 0)
            pltpu.async_copy(p_hbm.at[gidx_v], g_v, sem).wait()

            def add_row(r, _):
                for c in range(REL_D // 16):
                    g_v[r, pl.ds(c * 16, 16)] = (
                        g_v[r, pl.ds(c * 16, 16)] + a_v[r, pl.ds(c * 16, 16)]
                    )
                return 0

            lax.fori_loop(0, JH, add_row, 0)
            row0 = b * SEG * SEG + i_g * SEG + h * JH
            pltpu.sync_copy(g_v, out_hbm.at[pl.ds(row0, JH)])
            return 0

        lax.fori_loop(0, I_PER_W, per_i, 0)
        return 0

    lax.fori_loop(0, 2, do_half, 0)


def _mm_body(x_ref, w_ref, o_ref):
    o_ref[...] = jnp.dot(x_ref[...], w_ref[...],
                         preferred_element_type=jnp.float32)


def _mm_bias_body(x_ref, w_ref, b_ref, o_ref):
    o_ref[...] = jnp.dot(x_ref[...], w_ref[...],
                         preferred_element_type=jnp.float32) + b_ref[...]


_P_ROWS = TOTAL * SRC_LEN
_P_BLK = 4096

_p_matmul = pl.pallas_call(
    _mm_body,
    grid=(_P_ROWS // _P_BLK,),
    in_specs=[
        pl.BlockSpec((_P_BLK, RNN_D), lambda i: (i, 0)),
        pl.BlockSpec((RNN_D, REL_D), lambda i: (0, 0)),
    ],
    out_specs=pl.BlockSpec((_P_BLK, REL_D), lambda i: (i, 0)),
    out_shape=jax.ShapeDtypeStruct((_P_ROWS, REL_D), jnp.float32),
)

_small_matmul = pl.pallas_call(
    _mm_bias_body,
    in_specs=[
        pl.BlockSpec((TOTAL, IN_D), lambda: (0, 0)),
        pl.BlockSpec((IN_D, 2 * REL_D), lambda: (0, 0)),
        pl.BlockSpec((1, 2 * REL_D), lambda: (0, 0)),
    ],
    out_specs=pl.BlockSpec((TOTAL, 2 * REL_D), lambda: (0, 0)),
    out_shape=jax.ShapeDtypeStruct((TOTAL, 2 * REL_D), jnp.float32),
)


def kernel(input_tokens, index, src_enc, cu_seqlens, cat_table, lemma_table,
           head_W, head_b, dep_W, dep_b):
    del cu_seqlens
    cat_ids = input_tokens[:, 0]
    lem_ids = input_tokens[:, 1]
    src_flat = src_enc.reshape(TOTAL * SRC_LEN, RNN_D)
    head_idx = jnp.arange(TOTAL, dtype=jnp.int32) * SRC_LEN + index

    cat_pad = jnp.pad(cat_table, ((0, 0), (0, 128 - CAT_D)))
    cat_rows, lem_rows, head_rows = _sc_gather(
        cat_ids, lem_ids, head_idx, cat_pad, lemma_table, src_flat)
    amr_emb = jnp.concatenate([cat_rows[:, :CAT_D], lem_rows], axis=1)

    x = jnp.concatenate([amr_emb, head_rows], axis=1)
    w_cat = jnp.concatenate([
        head_W,
        jnp.concatenate([dep_W[:AMR_D],
                         jnp.zeros((RNN_D, REL_D), jnp.float32)], axis=0),
    ], axis=1)
    b_cat = jnp.concatenate([head_b, dep_b])[None, :]
    y = _small_matmul(x, w_cat, b_cat)
    head_out = y[:, :REL_D]
    a_term = y[:, REL_D:]

    p_rows = _p_matmul(src_flat, dep_W[AMR_D:])
    dep_flat = _sc_assemble(p_rows, a_term, index)
    return (amr_emb, head_out, dep_flat)


# speedup vs baseline: 2.2255x; 1.3785x over previous
import functools

import jax
import jax.numpy as jnp
from jax import lax
from jax.experimental import pallas as pl
from jax.experimental.pallas import tpu as pltpu
from jax.experimental.pallas import tpu_sc as plsc

B = 4
SEG = 256
TOTAL = B * SEG
SRC_LEN = 128
CAT_D = 32
LEM_D = 128
RNN_D = 128
REL_D = 256
AMR_D = CAT_D + LEM_D
IN_D = AMR_D + RNN_D

NC = 2
NS = 16
NW = NC * NS

_MESH = plsc.VectorSubcoreMesh(core_axis_name="c", subcore_axis_name="s")


def _wid():
    return lax.axis_index("s") * NC + lax.axis_index("c")


TOK_PER_W = TOTAL // NW


@functools.partial(
    pl.kernel,
    out_type=(
        jax.ShapeDtypeStruct((TOTAL, 128), jnp.float32),
        jax.ShapeDtypeStruct((TOTAL, LEM_D), jnp.float32),
        jax.ShapeDtypeStruct((TOTAL, RNN_D), jnp.float32),
    ),
    mesh=_MESH,
    scratch_types=[
        pltpu.VMEM((TOK_PER_W,), jnp.int32),
        pltpu.VMEM((TOK_PER_W,), jnp.int32),
        pltpu.VMEM((TOK_PER_W,), jnp.int32),
        pltpu.VMEM((TOK_PER_W, 128), jnp.float32),
        pltpu.VMEM((TOK_PER_W, LEM_D), jnp.float32),
        pltpu.VMEM((TOK_PER_W, RNN_D), jnp.float32),
        pltpu.SemaphoreType.DMA,
    ],
)
def _sc_gather(cat_ids, lem_ids, head_idx, cat_table, lemma_table, src_flat,
               cat_out, lem_out, head_out,
               cidx_v, lidx_v, hidx_v, crows_v, lrows_v, hrows_v, sem):
    base = _wid() * TOK_PER_W
    pltpu.sync_copy(cat_ids.at[pl.ds(base, TOK_PER_W)], cidx_v)
    pltpu.sync_copy(lem_ids.at[pl.ds(base, TOK_PER_W)], lidx_v)
    pltpu.sync_copy(head_idx.at[pl.ds(base, TOK_PER_W)], hidx_v)
    pltpu.async_copy(cat_table.at[cidx_v], crows_v, sem).wait()
    pltpu.sync_copy(crows_v, cat_out.at[pl.ds(base, TOK_PER_W)])
    pltpu.async_copy(lemma_table.at[lidx_v], lrows_v, sem).wait()
    pltpu.sync_copy(lrows_v, lem_out.at[pl.ds(base, TOK_PER_W)])
    pltpu.async_copy(src_flat.at[hidx_v], hrows_v, sem).wait()
    pltpu.sync_copy(hrows_v, head_out.at[pl.ds(base, TOK_PER_W)])


I_PER_W = SEG // 8
JH = SEG // 2


@functools.partial(
    pl.kernel,
    out_type=jax.ShapeDtypeStruct((B * SEG * SEG, REL_D), jnp.float32),
    mesh=_MESH,
    scratch_types=[
        pltpu.VMEM((SEG,), jnp.int32),
        pltpu.VMEM((JH,), jnp.int32),
        pltpu.VMEM((JH,), jnp.int32),
        pltpu.VMEM((JH, REL_D), jnp.float32),
        pltpu.VMEM((JH, REL_D), jnp.float32),
        pltpu.VMEM((JH, REL_D), jnp.float32),
        pltpu.SemaphoreType.DMA,
        pltpu.SemaphoreType.DMA,
        pltpu.SemaphoreType.DMA,
        pltpu.SemaphoreType.DMA,
    ],
)
def _sc_assemble(p_hbm, a_hbm, idx_hbm, out_hbm,
                 idx_v, gidx0, gidx1, a_v, g0, g1,
                 gsem0, gsem1, osem0, osem1):
    w = _wid()
    b = w // 8
    iblk = w % 8
    gidx = (gidx0, gidx1)
    g = (g0, g1)
    gsem = (gsem0, gsem1)
    osem = (osem0, osem1)
    pltpu.sync_copy(idx_hbm.at[pl.ds(b * SEG, SEG)], idx_v)

    def build_gidx(buf, h, i):
        i_g = iblk * I_PER_W + i
        off = (b * SEG + i_g) * SRC_LEN
        for c in range(JH // 16):
            buf[pl.ds(c * 16, 16)] = idx_v[pl.ds(h * JH + c * 16, 16)] + off

    def start_gather(bsel, h, i):
        build_gidx(gidx[bsel], h, i)
        pltpu.async_copy(p_hbm.at[gidx[bsel]], g[bsel], gsem[bsel])

    def wait_gather(bsel):
        pltpu.make_async_copy(p_hbm.at[gidx[bsel]], g[bsel], gsem[bsel]).wait()

    def start_out(bsel, h, i):
        i_g = iblk * I_PER_W + i
        row0 = b * SEG * SEG + i_g * SEG + h * JH
        pltpu.async_copy(g[bsel], out_hbm.at[pl.ds(row0, JH)], osem[bsel])

    def wait_out(bsel):
        pltpu.make_async_copy(g[bsel], out_hbm.at[pl.ds(0, JH)],
                              osem[bsel]).wait()

    def do_half(h, _):
        pltpu.sync_copy(a_hbm.at[pl.ds(b * SEG + h * JH, JH)], a_v)
        start_gather(0, h, 0)

        def pair(p, _):
            for bsel in (0, 1):
                i = 2 * p + bsel
                nb = 1 - bsel

                @pl.when(i + 1 < I_PER_W)
                def _():
                    pl.when(i >= 1)(lambda: wait_out(nb))
                    start_gather(nb, h, i + 1)

                wait_gather(bsel)

                @plsc.parallel_loop(0, JH, step=1, unroll=4)
                def _(r):
                    gb = g[bsel]
                    for c in range(REL_D // 16):
                        gb[r, pl.ds(c * 16, 16)] = (
                            gb[r, pl.ds(c * 16, 16)] + a_v[r, pl.ds(c * 16, 16)]
                        )

                start_out(bsel, h, i)
            return 0

        lax.fori_loop(0, I_PER_W // 2, pair, 0)
        wait_out(0)
        wait_out(1)
        return 0

    lax.fori_loop(0, 2, do_half, 0)


def _mm_body(x_ref, w_ref, o_ref):
    o_ref[...] = jnp.dot(x_ref[...], w_ref[...],
                         preferred_element_type=jnp.float32)


def _mm_bias_body(x_ref, w_ref, b_ref, o_ref):
    o_ref[...] = jnp.dot(x_ref[...], w_ref[...],
                         preferred_element_type=jnp.float32) + b_ref[...]


_P_ROWS = TOTAL * SRC_LEN
_P_BLK = 4096

_p_matmul = pl.pallas_call(
    _mm_body,
    grid=(_P_ROWS // _P_BLK,),
    in_specs=[
        pl.BlockSpec((_P_BLK, RNN_D), lambda i: (i, 0)),
        pl.BlockSpec((RNN_D, REL_D), lambda i: (0, 0)),
    ],
    out_specs=pl.BlockSpec((_P_BLK, REL_D), lambda i: (i, 0)),
    out_shape=jax.ShapeDtypeStruct((_P_ROWS, REL_D), jnp.float32),
)

_small_matmul = pl.pallas_call(
    _mm_bias_body,
    in_specs=[
        pl.BlockSpec((TOTAL, IN_D), lambda: (0, 0)),
        pl.BlockSpec((IN_D, 2 * REL_D), lambda: (0, 0)),
        pl.BlockSpec((1, 2 * REL_D), lambda: (0, 0)),
    ],
    out_specs=pl.BlockSpec((TOTAL, 2 * REL_D), lambda: (0, 0)),
    out_shape=jax.ShapeDtypeStruct((TOTAL, 2 * REL_D), jnp.float32),
)


def kernel(input_tokens, index, src_enc, cu_seqlens, cat_table, lemma_table,
           head_W, head_b, dep_W, dep_b):
    del cu_seqlens
    cat_ids = input_tokens[:, 0]
    lem_ids = input_tokens[:, 1]
    src_flat = src_enc.reshape(TOTAL * SRC_LEN, RNN_D)
    head_idx = jnp.arange(TOTAL, dtype=jnp.int32) * SRC_LEN + index

    cat_pad = jnp.pad(cat_table, ((0, 0), (0, 128 - CAT_D)))
    cat_rows, lem_rows, head_rows = _sc_gather(
        cat_ids, lem_ids, head_idx, cat_pad, lemma_table, src_flat)
    amr_emb = jnp.concatenate([cat_rows[:, :CAT_D], lem_rows], axis=1)

    x = jnp.concatenate([amr_emb, head_rows], axis=1)
    w_cat = jnp.concatenate([
        head_W,
        jnp.concatenate([dep_W[:AMR_D],
                         jnp.zeros((RNN_D, REL_D), jnp.float32)], axis=0),
    ], axis=1)
    b_cat = jnp.concatenate([head_b, dep_b])[None, :]
    y = _small_matmul(x, w_cat, b_cat)
    head_out = y[:, :REL_D]
    a_term = y[:, REL_D:]

    p_rows = _p_matmul(src_flat, dep_W[AMR_D:])
    dep_flat = _sc_assemble(p_rows, a_term, index)
    return (amr_emb, head_out, dep_flat)
